# baseline (speedup 1.0000x reference)
import functools

import jax
import jax.numpy as jnp
from jax import lax
from jax.experimental import pallas as pl
from jax.experimental.pallas import tpu as pltpu
from jax.experimental.pallas import tpu_sc as plsc

_B, _S, _D = 4, 2048, 2048
_NUM_NODES = 32
_NODE_DIM = _D // _NUM_NODES
_K = 8
_HID = _NODE_DIM // 2
_T = 512

_N_TOK = _B * _S
_ROWS = _N_TOK * _K
_PAIR_W = 2 * _NODE_DIM
_NC, _NS = 2, 16
_NW = _NC * _NS
_RPW = _ROWS // _NW
_CH = 128
_NCH = _RPW // _CH


def _score_kernel(x2_ref, w1_ref, b1_ref, w2_ref, idx_ref, pair_ref, *, tok_base):
    x2 = x2_ref[...]
    t = x2.shape[0]
    h = jnp.dot(
        x2.astype(jnp.bfloat16),
        w1_ref[...],
        preferred_element_type=jnp.float32,
    )
    h = h + b1_ref[...]
    h = 0.5 * h * (1.0 + lax.erf(h * 0.7071067811865476))
    s = jnp.dot(
        h.astype(jnp.bfloat16),
        w2_ref[...],
        preferred_element_type=jnp.float32,
    )

    iota = lax.broadcasted_iota(jnp.int32, (t, _NUM_NODES), 1)
    cur = s
    idxs = []
    for _ in range(_K):
        m = jnp.max(cur, axis=1, keepdims=True)
        idx_k = jnp.min(
            jnp.where(cur == m, iota, _NUM_NODES), axis=1, keepdims=True
        )
        cur = jnp.where(iota == idx_k, -jnp.inf, cur)
        idxs.append(idx_k)

    idx = jnp.concatenate(idxs, axis=1)
    tok = (
        tok_base
        + pl.program_id(0) * t
        + lax.broadcasted_iota(jnp.int32, (t, _K), 0)
    )
    flat = tok * _NUM_NODES + idx
    idx_ref[...] = flat
    pair_ref[...] = jnp.transpose(flat >> 1)


def _sc_gather_kernel(table_hbm, pair_hbm, out_hbm, idx_v, rows_v, sem):
    wid = lax.axis_index("s") * _NC + lax.axis_index("c")
    rpw = out_hbm.shape[0] // _NW
    base = wid * rpw

    def body(i, carry):
        off = base + i * _CH
        pltpu.sync_copy(pair_hbm.at[pl.ds(off, _CH)], idx_v)
        pltpu.async_copy(table_hbm.at[idx_v], rows_v, sem).wait()
        pltpu.sync_copy(rows_v, out_hbm.at[pl.ds(off, _CH)])
        return carry

    lax.fori_loop(0, rpw // _CH, body, 0)


def _proj_kernel(raw_ref, idx_ref, wp_ref, bp_ref, out_ref):
    raw = raw_ref[...]
    t = raw.shape[1]
    parity = idx_ref[...] & 1
    parts = []
    for k in range(_K):
        raw_k = raw[k]
        par_k = parity[:, k : k + 1]
        parts.append(
            jnp.where(par_k == 1, raw_k[:, _NODE_DIM:], raw_k[:, :_NODE_DIM])
        )
    pooled = jnp.concatenate(parts, axis=1)
    out_ref[...] = (
        jnp.dot(pooled, wp_ref[...], preferred_element_type=jnp.float32)
        + bp_ref[...]
    )


@jax.jit
def kernel(graph_features, W1, b1, W2, b2, Wp, bp):
    del b2
    bsz, seqlen, dmodel = graph_features.shape
    n_tok = bsz * seqlen
    x2 = graph_features.reshape(n_tok, _D)

    eye = jnp.eye(_NUM_NODES, dtype=jnp.float32)
    w1bd = (eye[:, None, :, None] * W1[None, :, None, :]).reshape(
        _D, _NUM_NODES * _HID
    ).astype(jnp.bfloat16)
    w2bd = (eye[:, None, :] * W2[None, :, 0, None]).reshape(
        _NUM_NODES * _HID, _NUM_NODES
    ).astype(jnp.bfloat16)
    b1bd = jnp.tile(b1, _NUM_NODES).reshape(1, _NUM_NODES * _HID)

    table = graph_features.reshape(n_tok * _NUM_NODES // 2, _PAIR_W)
    bp2 = bp.reshape(1, _D)

    n_chunk = 2
    ct = n_tok // n_chunk
    cb = ct // _T
    rows_c = ct * _K

    outs = []
    for c in range(n_chunk):
        flat_c, pair_c = pl.pallas_call(
            functools.partial(_score_kernel, tok_base=c * ct),
            grid=(cb,),
            in_specs=[
                pl.BlockSpec((_T, _D), lambda i, c=c: (c * cb + i, 0)),
                pl.BlockSpec((_D, _NUM_NODES * _HID), lambda i: (0, 0)),
                pl.BlockSpec((1, _NUM_NODES * _HID), lambda i: (0, 0)),
                pl.BlockSpec((_NUM_NODES * _HID, _NUM_NODES), lambda i: (0, 0)),
            ],
            out_specs=[
                pl.BlockSpec((_T, _K), lambda i: (i, 0)),
                pl.BlockSpec((_K, _T), lambda i: (0, i)),
            ],
            out_shape=[
                jax.ShapeDtypeStruct((ct, _K), jnp.int32),
                jax.ShapeDtypeStruct((_K, ct), jnp.int32),
            ],
            compiler_params=pltpu.CompilerParams(
                dimension_semantics=("parallel",)
            ),
        )(x2, w1bd, b1bd, w2bd)

        gather = functools.partial(
            pl.kernel,
            mesh=plsc.VectorSubcoreMesh(
                core_axis_name="c", subcore_axis_name="s"
            ),
            out_type=jax.ShapeDtypeStruct((rows_c, _PAIR_W), jnp.float32),
            scratch_types=[
                pltpu.VMEM((_CH,), jnp.int32),
                pltpu.VMEM((_CH, _PAIR_W), jnp.float32),
                pltpu.SemaphoreType.DMA,
            ],
        )(_sc_gather_kernel)
        raw = gather(table, pair_c.reshape(rows_c))

        raw3 = raw.reshape(_K, ct, _PAIR_W)
        out_c = pl.pallas_call(
            _proj_kernel,
            grid=(cb,),
            in_specs=[
                pl.BlockSpec((_K, _T, _PAIR_W), lambda i: (0, i, 0)),
                pl.BlockSpec((_T, _K), lambda i: (i, 0)),
                pl.BlockSpec((_K * _NODE_DIM, _D), lambda i: (0, 0)),
                pl.BlockSpec((1, _D), lambda i: (0, 0)),
            ],
            out_specs=pl.BlockSpec((_T, _D), lambda i: (i, 0)),
            out_shape=jax.ShapeDtypeStruct((ct, _D), jnp.float32),
            compiler_params=pltpu.CompilerParams(
                dimension_semantics=("parallel",)
            ),
        )(raw3, flat_c, Wp, bp2)
        outs.append(out_c)

    out = jnp.concatenate(outs, axis=0)
    return out.reshape(bsz, seqlen, dmodel)


# speedup vs baseline: 1.1017x; 1.1017x over previous
import functools

import jax
import jax.numpy as jnp
from jax import lax
from jax.experimental import pallas as pl
from jax.experimental.pallas import tpu as pltpu
from jax.experimental.pallas import tpu_sc as plsc

_B, _S, _D = 4, 2048, 2048
_NUM_NODES = 32
_NODE_DIM = _D // _NUM_NODES
_K = 8
_HID = _NODE_DIM // 2
_T = 1024

_N_TOK = _B * _S
_ROWS = _N_TOK * _K
_PAIR_W = 2 * _NODE_DIM
_NC, _NS = 2, 16
_NW = _NC * _NS
_RPW = _ROWS // _NW
_CH = 128
_NCH = _RPW // _CH


def _score_kernel(x2_ref, w1_ref, b1_ref, w2_ref, idx_ref, pair_ref):
    x2 = x2_ref[...]
    t = x2.shape[0]
    h = jnp.dot(
        x2.astype(jnp.bfloat16),
        w1_ref[...],
        preferred_element_type=jnp.float32,
    )
    h = h + b1_ref[...]
    h = 0.5 * h * (1.0 + lax.erf(h * 0.7071067811865476))
    s = jnp.dot(
        h.astype(jnp.bfloat16),
        w2_ref[...],
        preferred_element_type=jnp.float32,
    )

    iota = lax.broadcasted_iota(jnp.int32, (t, _NUM_NODES), 1)
    cur = s
    idxs = []
    for _ in range(_K):
        m = jnp.max(cur, axis=1, keepdims=True)
        idx_k = jnp.min(
            jnp.where(cur == m, iota, _NUM_NODES), axis=1, keepdims=True
        )
        cur = jnp.where(iota == idx_k, -jnp.inf, cur)
        idxs.append(idx_k)

    idx = jnp.concatenate(idxs, axis=1)
    tok = pl.program_id(0) * t + lax.broadcasted_iota(jnp.int32, (t, _K), 0)
    flat = tok * _NUM_NODES + idx
    idx_ref[...] = flat
    pair_ref[...] = jnp.transpose(flat >> 1)


def _sc_gather_kernel(table_hbm, pair_hbm, out_hbm, idx_v, rows_v, sem):
    wid = lax.axis_index("s") * _NC + lax.axis_index("c")
    base = wid * _RPW

    def body(i, carry):
        off = base + i * _CH
        pltpu.sync_copy(pair_hbm.at[pl.ds(off, _CH)], idx_v)
        pltpu.async_copy(table_hbm.at[idx_v], rows_v, sem).wait()
        pltpu.sync_copy(rows_v, out_hbm.at[pl.ds(off, _CH)])
        return carry

    lax.fori_loop(0, _NCH, body, 0)


def _proj_kernel(raw_ref, idx_ref, wp_ref, bp_ref, out_ref):
    raw = raw_ref[...]
    t = raw.shape[1]
    parity = idx_ref[...] & 1
    parts = []
    for k in range(_K):
        raw_k = raw[k]
        par_k = parity[:, k : k + 1]
        parts.append(
            jnp.where(par_k == 1, raw_k[:, _NODE_DIM:], raw_k[:, :_NODE_DIM])
        )
    pooled = jnp.concatenate(parts, axis=1)
    out_ref[...] = (
        jnp.dot(pooled, wp_ref[...], preferred_element_type=jnp.float32)
        + bp_ref[...]
    )


@jax.jit
def kernel(graph_features, W1, b1, W2, b2, Wp, bp):
    del b2
    bsz, seqlen, dmodel = graph_features.shape
    n_tok = bsz * seqlen
    x2 = graph_features.reshape(n_tok, _D)

    eye = jnp.eye(_NUM_NODES, dtype=jnp.float32)
    w1bd = (eye[:, None, :, None] * W1[None, :, None, :]).reshape(
        _D, _NUM_NODES * _HID
    ).astype(jnp.bfloat16)
    w2bd = (eye[:, None, :] * W2[None, :, 0, None]).reshape(
        _NUM_NODES * _HID, _NUM_NODES
    ).astype(jnp.bfloat16)
    b1bd = jnp.tile(b1, _NUM_NODES).reshape(1, _NUM_NODES * _HID)

    grid = (n_tok // _T,)
    flat_idx, pair_idx = pl.pallas_call(
        _score_kernel,
        grid=grid,
        in_specs=[
            pl.BlockSpec((_T, _D), lambda i: (i, 0)),
            pl.BlockSpec((_D, _NUM_NODES * _HID), lambda i: (0, 0)),
            pl.BlockSpec((1, _NUM_NODES * _HID), lambda i: (0, 0)),
            pl.BlockSpec((_NUM_NODES * _HID, _NUM_NODES), lambda i: (0, 0)),
        ],
        out_specs=[
            pl.BlockSpec((_T, _K), lambda i: (i, 0)),
            pl.BlockSpec((_K, _T), lambda i: (0, i)),
        ],
        out_shape=[
            jax.ShapeDtypeStruct((n_tok, _K), jnp.int32),
            jax.ShapeDtypeStruct((_K, n_tok), jnp.int32),
        ],
        compiler_params=pltpu.CompilerParams(
            dimension_semantics=("parallel",)
        ),
    )(x2, w1bd, b1bd, w2bd)

    table = graph_features.reshape(n_tok * _NUM_NODES // 2, _PAIR_W)
    gather = functools.partial(
        pl.kernel,
        mesh=plsc.VectorSubcoreMesh(core_axis_name="c", subcore_axis_name="s"),
        out_type=jax.ShapeDtypeStruct((_ROWS, _PAIR_W), jnp.float32),
        scratch_types=[
            pltpu.VMEM((_CH,), jnp.int32),
            pltpu.VMEM((_CH, _PAIR_W), jnp.float32),
            pltpu.SemaphoreType.DMA,
        ],
    )(_sc_gather_kernel)
    raw = gather(table, pair_idx.reshape(_ROWS))

    raw3 = raw.reshape(_K, n_tok, _PAIR_W)
    out = pl.pallas_call(
        _proj_kernel,
        grid=grid,
        in_specs=[
            pl.BlockSpec((_K, _T, _PAIR_W), lambda i: (0, i, 0)),
            pl.BlockSpec((_T, _K), lambda i: (i, 0)),
            pl.BlockSpec((_K * _NODE_DIM, _D), lambda i: (0, 0)),
            pl.BlockSpec((1, _D), lambda i: (0, 0)),
        ],
        out_specs=pl.BlockSpec((_T, _D), lambda i: (i, 0)),
        out_shape=jax.ShapeDtypeStruct((n_tok, _D), jnp.float32),
        compiler_params=pltpu.CompilerParams(
            dimension_semantics=("parallel",)
        ),
    )(raw3, flat_idx, Wp, bp.reshape(1, _D))
    return out.reshape(bsz, seqlen, dmodel)


# speedup vs baseline: 1.2362x; 1.1220x over previous
import functools

import jax
import jax.numpy as jnp
from jax import lax
from jax.experimental import pallas as pl
from jax.experimental.pallas import tpu as pltpu
from jax.experimental.pallas import tpu_sc as plsc

_B, _S, _D = 4, 2048, 2048
_NUM_NODES = 32
_NODE_DIM = _D // _NUM_NODES
_K = 8
_HID = _NODE_DIM // 2
_T = 1024

_N_TOK = _B * _S
_ROWS = _N_TOK * _K
_PAIR_W = 2 * _NODE_DIM
_NC, _NS = 2, 16
_NW = _NC * _NS
_RPW = _ROWS // _NW
_CH = 128
_NCH = _RPW // _CH


_G = 8
_GW = _G * _NODE_DIM
_GH = _G * _HID


def _score_kernel(x2_ref, w1_ref, b1_ref, w2_ref, idx_ref, pair_ref):
    x2 = x2_ref[...]
    t = x2.shape[0]
    ss = []
    for g in range(_NUM_NODES // _G):
        xg = x2[:, g * _GW : (g + 1) * _GW]
        hg = jnp.dot(
            xg.astype(jnp.bfloat16),
            w1_ref[...],
            preferred_element_type=jnp.float32,
        )
        hg = hg + b1_ref[...]
        hg = 0.5 * hg * (1.0 + lax.erf(hg * 0.7071067811865476))
        ss.append(
            jnp.dot(
                hg.astype(jnp.bfloat16),
                w2_ref[...],
                preferred_element_type=jnp.float32,
            )
        )
    s = jnp.concatenate(ss, axis=1)

    iota = lax.broadcasted_iota(jnp.int32, (t, _NUM_NODES), 1)
    cur = s
    idxs = []
    for _ in range(_K):
        m = jnp.max(cur, axis=1, keepdims=True)
        idx_k = jnp.min(
            jnp.where(cur == m, iota, _NUM_NODES), axis=1, keepdims=True
        )
        cur = jnp.where(iota == idx_k, -jnp.inf, cur)
        idxs.append(idx_k)

    idx = jnp.concatenate(idxs, axis=1)
    tok = pl.program_id(0) * t + lax.broadcasted_iota(jnp.int32, (t, _K), 0)
    flat = tok * _NUM_NODES + idx
    idx_ref[...] = flat
    pair_ref[...] = jnp.transpose(flat >> 1)


def _sc_gather_kernel(table_hbm, pair_hbm, out_hbm, idx_v, rows_v, sem):
    wid = lax.axis_index("s") * _NC + lax.axis_index("c")
    base = wid * _RPW

    def body(i, carry):
        off = base + i * _CH
        pltpu.sync_copy(pair_hbm.at[pl.ds(off, _CH)], idx_v)
        pltpu.async_copy(table_hbm.at[idx_v], rows_v, sem).wait()
        pltpu.sync_copy(rows_v, out_hbm.at[pl.ds(off, _CH)])
        return carry

    lax.fori_loop(0, _NCH, body, 0)


def _proj_kernel(raw_ref, idx_ref, wp_ref, bp_ref, out_ref):
    raw = raw_ref[...]
    t = raw.shape[1]
    parity = idx_ref[...] & 1
    parts = []
    for k in range(_K):
        raw_k = raw[k]
        par_k = parity[:, k : k + 1]
        parts.append(
            jnp.where(par_k == 1, raw_k[:, _NODE_DIM:], raw_k[:, :_NODE_DIM])
        )
    pooled = jnp.concatenate(parts, axis=1)
    out_ref[...] = (
        jnp.dot(pooled, wp_ref[...], preferred_element_type=jnp.float32)
        + bp_ref[...]
    )


@jax.jit
def kernel(graph_features, W1, b1, W2, b2, Wp, bp):
    del b2
    bsz, seqlen, dmodel = graph_features.shape
    n_tok = bsz * seqlen
    x2 = graph_features.reshape(n_tok, _D)

    eye = jnp.eye(_G, dtype=jnp.float32)
    w1bd = (eye[:, None, :, None] * W1[None, :, None, :]).reshape(
        _GW, _GH
    ).astype(jnp.bfloat16)
    w2bd = (eye[:, None, :] * W2[None, :, 0, None]).reshape(
        _GH, _G
    ).astype(jnp.bfloat16)
    b1bd = jnp.tile(b1, _G).reshape(1, _GH)

    grid = (n_tok // _T,)
    flat_idx, pair_idx = pl.pallas_call(
        _score_kernel,
        grid=grid,
        in_specs=[
            pl.BlockSpec((_T, _D), lambda i: (i, 0)),
            pl.BlockSpec((_GW, _GH), lambda i: (0, 0)),
            pl.BlockSpec((1, _GH), lambda i: (0, 0)),
            pl.BlockSpec((_GH, _G), lambda i: (0, 0)),
        ],
        out_specs=[
            pl.BlockSpec((_T, _K), lambda i: (i, 0)),
            pl.BlockSpec((_K, _T), lambda i: (0, i)),
        ],
        out_shape=[
            jax.ShapeDtypeStruct((n_tok, _K), jnp.int32),
            jax.ShapeDtypeStruct((_K, n_tok), jnp.int32),
        ],
        compiler_params=pltpu.CompilerParams(
            dimension_semantics=("parallel",)
        ),
    )(x2, w1bd, b1bd, w2bd)

    table = graph_features.reshape(n_tok * _NUM_NODES // 2, _PAIR_W)
    gather = functools.partial(
        pl.kernel,
        mesh=plsc.VectorSubcoreMesh(core_axis_name="c", subcore_axis_name="s"),
        out_type=jax.ShapeDtypeStruct((_ROWS, _PAIR_W), jnp.float32),
        scratch_types=[
            pltpu.VMEM((_CH,), jnp.int32),
            pltpu.VMEM((_CH, _PAIR_W), jnp.float32),
            pltpu.SemaphoreType.DMA,
        ],
    )(_sc_gather_kernel)
    raw = gather(table, pair_idx.reshape(_ROWS))

    raw3 = raw.reshape(_K, n_tok, _PAIR_W)
    out = pl.pallas_call(
        _proj_kernel,
        grid=grid,
        in_specs=[
            pl.BlockSpec((_K, _T, _PAIR_W), lambda i: (0, i, 0)),
            pl.BlockSpec((_T, _K), lambda i: (i, 0)),
            pl.BlockSpec((_K * _NODE_DIM, _D), lambda i: (0, 0)),
            pl.BlockSpec((1, _D), lambda i: (0, 0)),
        ],
        out_specs=pl.BlockSpec((_T, _D), lambda i: (i, 0)),
        out_shape=jax.ShapeDtypeStruct((n_tok, _D), jnp.float32),
        compiler_params=pltpu.CompilerParams(
            dimension_semantics=("parallel",)
        ),
    )(raw3, flat_idx, Wp, bp.reshape(1, _D))
    return out.reshape(bsz, seqlen, dmodel)
